# Initial kernel scaffold; baseline (speedup 1.0000x reference)
import jax
import jax.numpy as jnp
from jax.experimental import pallas as pl


def kernel(pos, s, edges, edges_h, reach_h, edge_index, pi, enc_W, enc_b, mp_W1, mp_b1, mp_W2, mp_b2, dec_W1, dec_b1, dec_W2, dec_b2):
    raise NotImplementedError("write your pallas kernel here")


# speedup vs baseline: 3.8693x; 3.8693x over previous
import functools

import jax
import jax.numpy as jnp
from jax import lax
from jax.experimental import pallas as pl
from jax.experimental.pallas import tpu as pltpu
from jax.experimental.pallas import tpu_sc as plsc

F32 = jnp.float32
I32 = jnp.int32

N = 10000
E = 160000
D = 128

NC = 2
NS = 16
NW = NC * NS
CH = 128

E_HI = 5008
E_LO = 4992
NFULL = 39
HI_BASE_END = 16 * E_HI


def _wid_base():
    c = lax.axis_index("c")
    s = lax.axis_index("s")
    wid = s * NC + c
    base = jnp.where(wid < 16, wid * E_HI, HI_BASE_END + (wid - 16) * E_LO)
    return c, s, wid, base


def _sc_mesh():
    return plsc.VectorSubcoreMesh(core_axis_name="c", subcore_axis_name="s")


def _k1_call(m, src, dst, zeros_nd):
    @functools.partial(
        pl.kernel,
        out_type=jax.ShapeDtypeStruct((NC, N, D), F32),
        mesh=_sc_mesh(),
        compiler_params=pltpu.CompilerParams(needs_layout_passes=False),
        scratch_types=[
            pltpu.VMEM((CH,), I32),
            pltpu.VMEM((CH,), I32),
            pltpu.VMEM((CH, D), F32),
            pltpu.VMEM((16,), I32),
            pltpu.VMEM((16,), I32),
            pltpu.VMEM((16, D), F32),
            pltpu.VMEM_SHARED((N, D), F32),
            pltpu.SemaphoreType.DMA,
        ],
    )
    def k1(m_hbm, src_hbm, dst_hbm, z_hbm, out_hbm,
           sidx, didx, rows, sidx16, didx16, rows16, acc, sem):
        c, s, wid, base = _wid_base()

        @pl.when(s < 15)
        def _():
            pltpu.sync_copy(z_hbm.at[pl.ds(s * 632, 632)],
                            acc.at[pl.ds(s * 632, 632)])

        @pl.when(s == 15)
        def _():
            pltpu.sync_copy(z_hbm.at[pl.ds(9480, 520)],
                            acc.at[pl.ds(9480, 520)])

        plsc.subcore_barrier()

        def chunk(k, carry):
            off = base + k * CH
            pltpu.sync_copy(src_hbm.at[pl.ds(off, CH)], sidx)
            pltpu.async_copy(m_hbm.at[sidx], rows, sem).wait()
            pltpu.sync_copy(dst_hbm.at[pl.ds(off, CH)], didx)
            pltpu.sync_copy(rows, acc.at[didx], add=True)
            return carry

        lax.fori_loop(0, NFULL, chunk, 0)

        @pl.when(wid < 16)
        def _():
            off = base + NFULL * CH
            pltpu.sync_copy(src_hbm.at[pl.ds(off, 16)], sidx16)
            pltpu.async_copy(m_hbm.at[sidx16], rows16, sem).wait()
            pltpu.sync_copy(dst_hbm.at[pl.ds(off, 16)], didx16)
            pltpu.sync_copy(rows16, acc.at[didx16], add=True)

        plsc.subcore_barrier()

        @pl.when(s < 15)
        def _():
            pltpu.sync_copy(acc.at[pl.ds(s * 632, 632)],
                            out_hbm.at[c, pl.ds(s * 632, 632)])

        @pl.when(s == 15)
        def _():
            pltpu.sync_copy(acc.at[pl.ds(9480, 520)],
                            out_hbm.at[c, pl.ds(9480, 520)])

    return k1(m, src, dst, zeros_nd)


def _k2_call(a, b, src, dst):
    @functools.partial(
        pl.kernel,
        out_type=(jax.ShapeDtypeStruct((E, D), F32),
                  jax.ShapeDtypeStruct((E, D), F32)),
        mesh=_sc_mesh(),
        compiler_params=pltpu.CompilerParams(needs_layout_passes=False),
        scratch_types=[
            pltpu.VMEM((CH,), I32),
            pltpu.VMEM((CH,), I32),
            pltpu.VMEM((CH, D), F32),
            pltpu.VMEM((CH, D), F32),
            pltpu.VMEM((16,), I32),
            pltpu.VMEM((16,), I32),
            pltpu.VMEM((16, D), F32),
            pltpu.VMEM((16, D), F32),
            pltpu.SemaphoreType.DMA,
            pltpu.SemaphoreType.DMA,
        ],
    )
    def k2(a_hbm, b_hbm, src_hbm, dst_hbm, ga_hbm, gb_hbm,
           sidx, didx, rowsa, rowsb, sidx16, didx16, rowsa16, rowsb16,
           sema, semb):
        c, s, wid, base = _wid_base()

        def chunk(k, carry):
            off = base + k * CH
            pltpu.sync_copy(src_hbm.at[pl.ds(off, CH)], sidx)
            cpa = pltpu.async_copy(a_hbm.at[sidx], rowsa, sema)
            pltpu.sync_copy(dst_hbm.at[pl.ds(off, CH)], didx)
            cpb = pltpu.async_copy(b_hbm.at[didx], rowsb, semb)
            cpa.wait()
            cpb.wait()
            pltpu.sync_copy(rowsa, ga_hbm.at[pl.ds(off, CH)])
            pltpu.sync_copy(rowsb, gb_hbm.at[pl.ds(off, CH)])
            return carry

        lax.fori_loop(0, NFULL, chunk, 0)

        @pl.when(wid < 16)
        def _():
            off = base + NFULL * CH
            pltpu.sync_copy(src_hbm.at[pl.ds(off, 16)], sidx16)
            cpa = pltpu.async_copy(a_hbm.at[sidx16], rowsa16, sema)
            pltpu.sync_copy(dst_hbm.at[pl.ds(off, 16)], didx16)
            cpb = pltpu.async_copy(b_hbm.at[didx16], rowsb16, semb)
            cpa.wait()
            cpb.wait()
            pltpu.sync_copy(rowsa16, ga_hbm.at[pl.ds(off, 16)])
            pltpu.sync_copy(rowsb16, gb_hbm.at[pl.ds(off, 16)])

    return k2(a, b, src, dst)


def _k3_call(alpha, src, dst, zeros_n):
    @functools.partial(
        pl.kernel,
        out_type=jax.ShapeDtypeStruct((NW, 1, N), F32),
        mesh=_sc_mesh(),
        compiler_params=pltpu.CompilerParams(needs_layout_passes=False),
        scratch_types=[
            pltpu.VMEM((CH,), F32),
            pltpu.VMEM((CH,), I32),
            pltpu.VMEM((CH,), I32),
            pltpu.VMEM((16,), F32),
            pltpu.VMEM((16,), I32),
            pltpu.VMEM((16,), I32),
            pltpu.VMEM((N,), F32),
        ],
    )
    def k3(a_hbm, src_hbm, dst_hbm, z_hbm, out_hbm,
           av, sidx, didx, av16, sidx16, didx16, cnt):
        c, s, wid, base = _wid_base()
        pltpu.sync_copy(z_hbm, cnt)

        ones = jnp.full((16,), 1.0, F32)

        def groups(avr, sr, dr, ng):
            for g in range(ng):
                a16 = avr[pl.ds(g * 16, 16)]
                m16 = a16 >= 0.4
                plsc.store_scatter(cnt, [sr[pl.ds(g * 16, 16)]], ones,
                                   mask=m16)
                plsc.store_scatter(cnt, [dr[pl.ds(g * 16, 16)]], ones,
                                   mask=m16)

        def chunk(k, carry):
            off = base + k * CH
            pltpu.sync_copy(a_hbm.at[pl.ds(off, CH)], av)
            pltpu.sync_copy(src_hbm.at[pl.ds(off, CH)], sidx)
            pltpu.sync_copy(dst_hbm.at[pl.ds(off, CH)], didx)
            groups(av, sidx, didx, CH // 16)
            return carry

        lax.fori_loop(0, NFULL, chunk, 0)

        @pl.when(wid < 16)
        def _():
            off = base + NFULL * CH
            pltpu.sync_copy(a_hbm.at[pl.ds(off, 16)], av16)
            pltpu.sync_copy(src_hbm.at[pl.ds(off, 16)], sidx16)
            pltpu.sync_copy(dst_hbm.at[pl.ds(off, 16)], didx16)
            groups(av16, sidx16, didx16, 1)

        pltpu.sync_copy(cnt, out_hbm.at[wid, 0])

    return k3(alpha, src, dst, zeros_n)


def _k4_call(alpha, src, dst, neg1f, neg1i):
    @functools.partial(
        pl.kernel,
        out_type=(jax.ShapeDtypeStruct((NW, 1, N), F32),
                  jax.ShapeDtypeStruct((NW, 1, N), I32)),
        mesh=_sc_mesh(),
        compiler_params=pltpu.CompilerParams(needs_layout_passes=False),
        scratch_types=[
            pltpu.VMEM((CH,), F32),
            pltpu.VMEM((CH,), I32),
            pltpu.VMEM((CH,), I32),
            pltpu.VMEM((16,), F32),
            pltpu.VMEM((16,), I32),
            pltpu.VMEM((16,), I32),
            pltpu.VMEM((N,), F32),
            pltpu.VMEM((N,), I32),
        ],
    )
    def k4(a_hbm, src_hbm, dst_hbm, nf_hbm, ni_hbm, bout_hbm, cout_hbm,
           av, sidx, didx, av16, sidx16, didx16, best, cand):
        c, s, wid, base = _wid_base()
        pltpu.sync_copy(nf_hbm, best)
        pltpu.sync_copy(ni_hbm, cand)
        lane = jnp.arange(16, dtype=I32)

        def groups(avr, sr, dr, ng):
            for g in range(ng):
                a16 = avr[pl.ds(g * 16, 16)]
                s16 = sr[pl.ds(g * 16, 16)]
                d16 = dr[pl.ds(g * 16, 16)]
                for j in range(16):
                    m = lane == j
                    b16 = plsc.load_gather(best, [d16])
                    c16 = plsc.load_gather(cand, [d16])
                    gt = a16 > b16
                    eq = a16 == b16
                    nb = jnp.where(gt, a16, b16)
                    ncd = jnp.where(gt, s16,
                                    jnp.where(eq, jnp.maximum(c16, s16), c16))
                    plsc.store_scatter(best, [d16], nb, mask=m)
                    plsc.store_scatter(cand, [d16], ncd, mask=m)

        def chunk(k, carry):
            off = base + k * CH
            pltpu.sync_copy(a_hbm.at[pl.ds(off, CH)], av)
            pltpu.sync_copy(src_hbm.at[pl.ds(off, CH)], sidx)
            pltpu.sync_copy(dst_hbm.at[pl.ds(off, CH)], didx)
            groups(av, sidx, didx, CH // 16)
            return carry

        lax.fori_loop(0, NFULL, chunk, 0)

        @pl.when(wid < 16)
        def _():
            off = base + NFULL * CH
            pltpu.sync_copy(a_hbm.at[pl.ds(off, 16)], av16)
            pltpu.sync_copy(src_hbm.at[pl.ds(off, 16)], sidx16)
            pltpu.sync_copy(dst_hbm.at[pl.ds(off, 16)], didx16)
            groups(av16, sidx16, didx16, 1)

        pltpu.sync_copy(best, bout_hbm.at[wid, 0])
        pltpu.sync_copy(cand, cout_hbm.at[wid, 0])

    return k4(alpha, src, dst, neg1f, neg1i)


BN = 1000
BE = 2000
BE2 = 4000
BN2 = 2000


def _dot(x, w):
    return jnp.dot(x, w, preferred_element_type=F32)


def _t1_call(pos2, x2, h, enc_W, enc_b, w1a, w1b, b1):
    def body(pos_r, x_r, h_r, ew_r, eb_r, w1a_r, w1b_r, b1_r, z_r, m_r):
        z = jnp.maximum(
            pos_r[...] * ew_r[0:1, :] + x_r[...] * ew_r[1:2, :] + eb_r[...],
            0.0)
        z_r[...] = z
        q = _dot(z, w1a_r[...]) + _dot(h_r[...], w1b_r[...])
        m_r[...] = jnp.maximum(q + b1_r[...], 0.0)

    full = lambda shape: pl.BlockSpec(shape, lambda i: (0, 0))
    return pl.pallas_call(
        body,
        grid=(N // BN,),
        in_specs=[
            pl.BlockSpec((BN, 1), lambda i: (i, 0)),
            pl.BlockSpec((BN, 1), lambda i: (i, 0)),
            pl.BlockSpec((BN, D), lambda i: (i, 0)),
            full((2, D)), full((1, D)), full((D, D)), full((D, D)),
            full((1, D)),
        ],
        out_specs=[pl.BlockSpec((BN, D), lambda i: (i, 0))] * 2,
        out_shape=[jax.ShapeDtypeStruct((N, D), F32)] * 2,
    )(pos2, x2, h, enc_W, enc_b, w1a, w1b, b1)


def _t2_call(z, h, a0, a1, w2a, w2b, b2, dw1a, dw1b):
    def body(z_r, h_r, a0_r, a1_r, w2a_r, w2b_r, b2_r, dw1a_r, dw1b_r,
             hn_r, A_r, B_r):
        acc = _dot(z_r[...], w2a_r[...]) + _dot(h_r[...], w2b_r[...])
        hn = jnp.maximum(acc + b2_r[...] + a0_r[...] + a1_r[...], 0.0)
        hn_r[...] = hn
        A_r[...] = _dot(hn, dw1a_r[...])
        B_r[...] = _dot(hn, dw1b_r[...])

    full = lambda shape: pl.BlockSpec(shape, lambda i: (0, 0))
    blk = pl.BlockSpec((BN, D), lambda i: (i, 0))
    return pl.pallas_call(
        body,
        grid=(N // BN,),
        in_specs=[blk, blk, blk, blk,
                  full((D, D)), full((D, D)), full((1, D)),
                  full((D, D)), full((D, D))],
        out_specs=[blk] * 3,
        out_shape=[jax.ShapeDtypeStruct((N, D), F32)] * 3,
    )(z, h, a0, a1, w2a, w2b, b2, dw1a, dw1b)


def _t3_call(ga, gb, db1, dw2, db2):
    def body(ga_r, gb_r, db1_r, dw2_r, db2_r, out_r):
        x = jnp.maximum(ga_r[...] + gb_r[...] + db1_r[...], 0.0)
        u = _dot(x, dw2_r[...]) + db2_r[...]
        out_r[...] = jax.nn.sigmoid(u)

    full = lambda shape: pl.BlockSpec(shape, lambda i: (0, 0))
    blk = pl.BlockSpec((BE, D), lambda i: (i, 0))
    return pl.pallas_call(
        body,
        grid=(E // BE,),
        in_specs=[blk, blk, full((1, D)), full((D, 1)), full((1, 1))],
        out_specs=pl.BlockSpec((BE, 1), lambda i: (i, 0)),
        out_shape=jax.ShapeDtypeStruct((E, 1), F32),
    )(ga, gb, db1, dw2, db2)


def _t4_call(cnt):
    def body(cnt_r, out_r):
        ssum = jnp.sum(cnt_r[...], axis=0, keepdims=True)
        out_r[...] = jnp.where(ssum > 0.0, 1.0, 0.0).astype(F32)

    return pl.pallas_call(
        body,
        out_shape=jax.ShapeDtypeStruct((1, N), F32),
    )(cnt)


def _t5e_call(a1, a2, a3, edges_r, h1, h2, h3):
    eps = 1e-7

    def body(a1_r, a2_r, a3_r, e_r, h1_r, h2_r, h3_r, acc_r):
        e = e_r[...]

        def bce_sum(a_blk, t_blk):
            p = jnp.clip(a_blk, eps, 1.0 - eps)
            return jnp.sum(-(t_blk * jnp.log(p)
                             + (1.0 - t_blk) * jnp.log(1.0 - p)))

        lx = bce_sum(a3_r[...], e)
        lh = (bce_sum(a1_r[...], h1_r[...])
              + bce_sum(a2_r[...], h2_r[...])
              + bce_sum(a3_r[...], h3_r[...]))

        def cnt_match(a_blk):
            pred = jnp.where(a_blk > 0.4, 1.0, 0.0).astype(F32)
            return jnp.sum(jnp.where(pred == e, 1.0, 0.0))

        ce = cnt_match(a1_r[...]) + cnt_match(a2_r[...]) + cnt_match(a3_r[...])
        lanei = lax.broadcasted_iota(I32, (1, 128), 1)
        vec = (jnp.where(lanei == 0, lx, 0.0)
               + jnp.where(lanei == 1, lh, 0.0)
               + jnp.where(lanei == 2, ce, 0.0))
        acc_r[...] = vec

    return pl.pallas_call(
        body,
        out_shape=jax.ShapeDtypeStruct((1, 128), F32),
    )(a1, a2, a3, edges_r, h1, h2, h3)


def _t5n_call(bp, cp, pi_r, reach_r, rh_r):
    def body(bp_r, cp_r, pi_rr, re_r, rh_rr, acc_r):
        bpv = bp_r[...]
        best = jnp.max(bpv, axis=0, keepdims=True)
        cand = jnp.max(jnp.where(bpv == best, cp_r[...], -1),
                       axis=0, keepdims=True)
        col = lax.broadcasted_iota(I32, (1, N), 1)
        parents = jnp.where(cand >= 0, cand, col)
        cpar = jnp.sum(jnp.where(parents == pi_rr[...], 1.0, 0.0))
        crch = jnp.sum(jnp.where(jnp.round(re_r[...]) == rh_rr[...], 1.0, 0.0))
        lanei = lax.broadcasted_iota(I32, (1, 128), 1)
        vec = (jnp.where(lanei == 0, cpar, 0.0)
               + jnp.where(lanei == 1, crch, 0.0))
        acc_r[...] = vec

    return pl.pallas_call(
        body,
        out_shape=jax.ShapeDtypeStruct((1, 128), F32),
    )(bp, cp, pi_r, reach_r, rh_r)


def kernel(pos, s, edges, edges_h, reach_h, edge_index, pi,
           enc_W, enc_b, mp_W1, mp_b1, mp_W2, mp_b2,
           dec_W1, dec_b1, dec_W2, dec_b2):
    src = edge_index[0]
    dst = edge_index[1]
    hints = edges_h[1:]
    T = hints.shape[0]

    w1a, w1b = mp_W1[:D], mp_W1[D:]
    w2a, w2b = mp_W2[:D], mp_W2[D:]
    dw1a, dw1b = dec_W1[:D], dec_W1[D:]
    enc_b_r = enc_b.reshape(1, D)
    b1_r = mp_b1.reshape(1, D)
    b2_r = mp_b2.reshape(1, D)
    db1_r = dec_b1.reshape(1, D)
    db2_r = dec_b2.reshape(1, 1)

    pos2 = pos.reshape(N, 1).astype(F32)
    x2 = s.reshape(N, 1).astype(F32)
    zeros_nd = jnp.zeros((N, D), F32)
    zeros_n = jnp.zeros((N,), F32)
    neg1f = jnp.full((N,), -1.0, F32)
    neg1i = jnp.full((N,), -1, I32)

    h = zeros_nd
    alphas = []
    reach_row = None
    for _ in range(T):
        z, m = _t1_call(pos2, x2, h, enc_W, enc_b_r, w1a, w1b, b1_r)
        aggp = _k1_call(m, src, dst, zeros_nd)
        h, a_mat, b_mat = _t2_call(z, h, aggp[0], aggp[1],
                                   w2a, w2b, b2_r, dw1a, dw1b)
        ga, gb = _k2_call(a_mat, b_mat, src, dst)
        alpha = _t3_call(ga, gb, db1_r, dec_W2, db2_r).reshape(E)
        alphas.append(alpha)
        cntp = _k3_call(alpha, src, dst, zeros_n)
        reach_row = _t4_call(cntp.reshape(NW, N))
        x2 = reach_row.reshape(N, 1)

    bp, cp = _k4_call(alphas[-1], src, dst, neg1f, neg1i)
    bp = bp.reshape(NW, N)
    cp = cp.reshape(NW, N)
    acc_e = _t5e_call(alphas[0].reshape(1, E), alphas[1].reshape(1, E),
                      alphas[2].reshape(1, E), edges.reshape(1, E),
                      hints[0].reshape(1, E), hints[1].reshape(1, E),
                      hints[2].reshape(1, E))
    acc_n = _t5n_call(bp, cp, pi.reshape(1, N), reach_row,
                      reach_h[-1].reshape(1, N))

    loss_x = acc_e[0, 0] / E
    loss_h = acc_e[0, 1] / E
    edges_err = acc_e[0, 2] / (E * T) * 100.0
    reach_err = acc_n[0, 1] / N * 100.0
    parents_err = acc_n[0, 0] / N
    return jnp.stack([loss_x, loss_h, edges_err, reach_err, parents_err])


# speedup vs baseline: 4.4336x; 1.1458x over previous
import functools

import jax
import jax.numpy as jnp
from jax import lax
from jax.experimental import pallas as pl
from jax.experimental.pallas import tpu as pltpu
from jax.experimental.pallas import tpu_sc as plsc

F32 = jnp.float32
I32 = jnp.int32

N = 10000
E = 160000
D = 128

NC = 2
NS = 16
NW = NC * NS
CH = 128

E_HI = 5008
E_LO = 4992
NFULL = 39
HI_BASE_END = 16 * E_HI

def _chunk_start(wid):
    return jnp.where(wid < 2, wid * 40, 80 + (wid - 2) * 39)


def _wid_base():
    c = lax.axis_index("c")
    s = lax.axis_index("s")
    wid = s * NC + c
    base = jnp.where(wid < 16, wid * E_HI, HI_BASE_END + (wid - 16) * E_LO)
    return c, s, wid, base


def _sc_mesh():
    return plsc.VectorSubcoreMesh(core_axis_name="c", subcore_axis_name="s")


def _k1_call(m, src, dst, zeros_nd):
    @functools.partial(
        pl.kernel,
        out_type=jax.ShapeDtypeStruct((NC, N, D), F32),
        mesh=_sc_mesh(),
        compiler_params=pltpu.CompilerParams(needs_layout_passes=False),
        scratch_types=[
            pltpu.VMEM((CH,), I32),
            pltpu.VMEM((CH,), I32),
            pltpu.VMEM((CH,), I32),
            pltpu.VMEM((CH,), I32),
            pltpu.VMEM((2, CH, D), F32),
            pltpu.VMEM_SHARED((N, D), F32),
            pltpu.SemaphoreType.DMA,
            pltpu.SemaphoreType.DMA,
            pltpu.SemaphoreType.DMA,
            pltpu.SemaphoreType.DMA,
        ],
    )
    def k1(m_hbm, src_hbm, dst_hbm, z_hbm, out_hbm,
           sidx0, sidx1, didx0, didx1, rows, acc, sg0, sg1, sa0, sa1):
        c, s, wid, base = _wid_base()
        sidx = (sidx0, sidx1)
        didx = (didx0, didx1)
        sg = (sg0, sg1)
        sa = (sa0, sa1)

        @pl.when(s < 15)
        def _():
            pltpu.sync_copy(z_hbm.at[pl.ds(s * 632, 632)],
                            acc.at[pl.ds(s * 632, 632)])

        @pl.when(s == 15)
        def _():
            pltpu.sync_copy(z_hbm.at[pl.ds(9480, 520)],
                            acc.at[pl.ds(9480, 520)])

        plsc.subcore_barrier()

        ebase = _chunk_start(wid) * CH
        gather_h = [None, None]
        add_h = [None, None]

        def load_idx(k, sl):
            off = ebase + k * CH
            pltpu.sync_copy(src_hbm.at[pl.ds(off, CH)], sidx[sl])
            pltpu.sync_copy(dst_hbm.at[pl.ds(off, CH)], didx[sl])

        load_idx(0, 0)
        gather_h[0] = pltpu.async_copy(m_hbm.at[sidx[0]], rows.at[0], sg[0])
        for k in range(NFULL):
            sl = k % 2
            ns = 1 - sl
            if k + 1 < NFULL:
                if add_h[ns] is not None:
                    add_h[ns].wait()
                load_idx(k + 1, ns)
                gather_h[ns] = pltpu.async_copy(
                    m_hbm.at[sidx[ns]], rows.at[ns], sg[ns])
            gather_h[sl].wait()
            add_h[sl] = pltpu.async_copy(
                rows.at[sl], acc.at[didx[sl]], sa[sl], add=True)
        for hnd in add_h:
            if hnd is not None:
                hnd.wait()

        @pl.when(wid < 2)
        def _():
            off = ebase + NFULL * CH
            pltpu.sync_copy(src_hbm.at[pl.ds(off, CH)], sidx0)
            pltpu.async_copy(m_hbm.at[sidx0], rows.at[0], sg0).wait()
            pltpu.sync_copy(dst_hbm.at[pl.ds(off, CH)], didx0)
            pltpu.async_copy(rows.at[0], acc.at[didx0], sa0, add=True).wait()

        plsc.subcore_barrier()

        @pl.when(s < 15)
        def _():
            pltpu.sync_copy(acc.at[pl.ds(s * 632, 632)],
                            out_hbm.at[c, pl.ds(s * 632, 632)])

        @pl.when(s == 15)
        def _():
            pltpu.sync_copy(acc.at[pl.ds(9480, 520)],
                            out_hbm.at[c, pl.ds(9480, 520)])

    return k1(m, src, dst, zeros_nd)


def _k2_call(a, b, src, dst):
    @functools.partial(
        pl.kernel,
        out_type=(jax.ShapeDtypeStruct((E, D), F32),
                  jax.ShapeDtypeStruct((E, D), F32)),
        mesh=_sc_mesh(),
        compiler_params=pltpu.CompilerParams(needs_layout_passes=False),
        scratch_types=[
            pltpu.VMEM((CH,), I32),
            pltpu.VMEM((CH,), I32),
            pltpu.VMEM((CH,), I32),
            pltpu.VMEM((CH,), I32),
            pltpu.VMEM((2, CH, D), F32),
            pltpu.VMEM((2, CH, D), F32),
            pltpu.SemaphoreType.DMA,
            pltpu.SemaphoreType.DMA,
            pltpu.SemaphoreType.DMA,
            pltpu.SemaphoreType.DMA,
            pltpu.SemaphoreType.DMA,
            pltpu.SemaphoreType.DMA,
            pltpu.SemaphoreType.DMA,
            pltpu.SemaphoreType.DMA,
        ],
    )
    def k2(a_hbm, b_hbm, src_hbm, dst_hbm, ga_hbm, gb_hbm,
           sidx0, sidx1, didx0, didx1, rowsa, rowsb,
           sga0, sga1, sgb0, sgb1, swa0, swa1, swb0, swb1):
        c, s, wid, base = _wid_base()
        sidx = (sidx0, sidx1)
        didx = (didx0, didx1)
        sga = (sga0, sga1)
        sgb = (sgb0, sgb1)
        swa = (swa0, swa1)
        swb = (swb0, swb1)
        ebase = _chunk_start(wid) * CH
        ga_h = [None, None]
        gb_h = [None, None]
        wa_h = [None, None]
        wb_h = [None, None]

        def load_idx(k, sl):
            off = ebase + k * CH
            pltpu.sync_copy(src_hbm.at[pl.ds(off, CH)], sidx[sl])
            pltpu.sync_copy(dst_hbm.at[pl.ds(off, CH)], didx[sl])

        def issue_gathers(k, sl):
            ga_h[sl] = pltpu.async_copy(a_hbm.at[sidx[sl]], rowsa.at[sl],
                                        sga[sl])
            gb_h[sl] = pltpu.async_copy(b_hbm.at[didx[sl]], rowsb.at[sl],
                                        sgb[sl])

        load_idx(0, 0)
        issue_gathers(0, 0)
        for k in range(NFULL):
            sl = k % 2
            ns = 1 - sl
            if k + 1 < NFULL:
                if wa_h[ns] is not None:
                    wa_h[ns].wait()
                    wb_h[ns].wait()
                load_idx(k + 1, ns)
                issue_gathers(k + 1, ns)
            ga_h[sl].wait()
            gb_h[sl].wait()
            off = ebase + k * CH
            wa_h[sl] = pltpu.async_copy(rowsa.at[sl],
                                        ga_hbm.at[pl.ds(off, CH)], swa[sl])
            wb_h[sl] = pltpu.async_copy(rowsb.at[sl],
                                        gb_hbm.at[pl.ds(off, CH)], swb[sl])
        for hnd in wa_h + wb_h:
            if hnd is not None:
                hnd.wait()

        @pl.when(wid < 2)
        def _():
            off = ebase + NFULL * CH
            pltpu.sync_copy(src_hbm.at[pl.ds(off, CH)], sidx0)
            pltpu.async_copy(a_hbm.at[sidx0], rowsa.at[0], sga0).wait()
            pltpu.sync_copy(dst_hbm.at[pl.ds(off, CH)], didx0)
            pltpu.async_copy(b_hbm.at[didx0], rowsb.at[0], sgb0).wait()
            pltpu.sync_copy(rowsa.at[0], ga_hbm.at[pl.ds(off, CH)])
            pltpu.sync_copy(rowsb.at[0], gb_hbm.at[pl.ds(off, CH)])

    return k2(a, b, src, dst)


def _k3_call(alpha, src, dst, zeros_n):
    @functools.partial(
        pl.kernel,
        out_type=jax.ShapeDtypeStruct((NW, 1, N), F32),
        mesh=_sc_mesh(),
        compiler_params=pltpu.CompilerParams(needs_layout_passes=False),
        scratch_types=[
            pltpu.VMEM((CH,), F32),
            pltpu.VMEM((CH,), I32),
            pltpu.VMEM((CH,), I32),
            pltpu.VMEM((16,), F32),
            pltpu.VMEM((16,), I32),
            pltpu.VMEM((16,), I32),
            pltpu.VMEM((N,), F32),
        ],
    )
    def k3(a_hbm, src_hbm, dst_hbm, z_hbm, out_hbm,
           av, sidx, didx, av16, sidx16, didx16, cnt):
        c, s, wid, base = _wid_base()
        pltpu.sync_copy(z_hbm, cnt)

        ones = jnp.full((16,), 1.0, F32)

        def groups(avr, sr, dr, ng):
            for g in range(ng):
                a16 = avr[pl.ds(g * 16, 16)]
                m16 = a16 >= 0.4
                plsc.store_scatter(cnt, [sr[pl.ds(g * 16, 16)]], ones,
                                   mask=m16)
                plsc.store_scatter(cnt, [dr[pl.ds(g * 16, 16)]], ones,
                                   mask=m16)

        def chunk(k, carry):
            off = base + k * CH
            pltpu.sync_copy(a_hbm.at[pl.ds(off, CH)], av)
            pltpu.sync_copy(src_hbm.at[pl.ds(off, CH)], sidx)
            pltpu.sync_copy(dst_hbm.at[pl.ds(off, CH)], didx)
            groups(av, sidx, didx, CH // 16)
            return carry

        lax.fori_loop(0, NFULL, chunk, 0)

        @pl.when(wid < 16)
        def _():
            off = base + NFULL * CH
            pltpu.sync_copy(a_hbm.at[pl.ds(off, 16)], av16)
            pltpu.sync_copy(src_hbm.at[pl.ds(off, 16)], sidx16)
            pltpu.sync_copy(dst_hbm.at[pl.ds(off, 16)], didx16)
            groups(av16, sidx16, didx16, 1)

        pltpu.sync_copy(cnt, out_hbm.at[wid, 0])

    return k3(alpha, src, dst, zeros_n)


def _k4_call(alpha, src, dst, neg1f, neg1i):
    @functools.partial(
        pl.kernel,
        out_type=(jax.ShapeDtypeStruct((NW, 1, N), F32),
                  jax.ShapeDtypeStruct((NW, 1, N), I32)),
        mesh=_sc_mesh(),
        compiler_params=pltpu.CompilerParams(needs_layout_passes=False),
        scratch_types=[
            pltpu.VMEM((CH,), F32),
            pltpu.VMEM((CH,), I32),
            pltpu.VMEM((CH,), I32),
            pltpu.VMEM((16,), F32),
            pltpu.VMEM((16,), I32),
            pltpu.VMEM((16,), I32),
            pltpu.VMEM((N,), F32),
            pltpu.VMEM((N,), I32),
        ],
    )
    def k4(a_hbm, src_hbm, dst_hbm, nf_hbm, ni_hbm, bout_hbm, cout_hbm,
           av, sidx, didx, av16, sidx16, didx16, best, cand):
        c, s, wid, base = _wid_base()
        pltpu.sync_copy(nf_hbm, best)
        pltpu.sync_copy(ni_hbm, cand)
        lane = jnp.arange(16, dtype=I32)

        def groups(avr, sr, dr, ng):
            for g in range(ng):
                a16 = avr[pl.ds(g * 16, 16)]
                s16 = sr[pl.ds(g * 16, 16)]
                d16 = dr[pl.ds(g * 16, 16)]
                for j in range(16):
                    m = lane == j
                    b16 = plsc.load_gather(best, [d16])
                    c16 = plsc.load_gather(cand, [d16])
                    gt = a16 > b16
                    eq = a16 == b16
                    nb = jnp.where(gt, a16, b16)
                    ncd = jnp.where(gt, s16,
                                    jnp.where(eq, jnp.maximum(c16, s16), c16))
                    plsc.store_scatter(best, [d16], nb, mask=m)
                    plsc.store_scatter(cand, [d16], ncd, mask=m)

        def chunk(k, carry):
            off = base + k * CH
            pltpu.sync_copy(a_hbm.at[pl.ds(off, CH)], av)
            pltpu.sync_copy(src_hbm.at[pl.ds(off, CH)], sidx)
            pltpu.sync_copy(dst_hbm.at[pl.ds(off, CH)], didx)
            groups(av, sidx, didx, CH // 16)
            return carry

        lax.fori_loop(0, NFULL, chunk, 0)

        @pl.when(wid < 16)
        def _():
            off = base + NFULL * CH
            pltpu.sync_copy(a_hbm.at[pl.ds(off, 16)], av16)
            pltpu.sync_copy(src_hbm.at[pl.ds(off, 16)], sidx16)
            pltpu.sync_copy(dst_hbm.at[pl.ds(off, 16)], didx16)
            groups(av16, sidx16, didx16, 1)

        pltpu.sync_copy(best, bout_hbm.at[wid, 0])
        pltpu.sync_copy(cand, cout_hbm.at[wid, 0])

    return k4(alpha, src, dst, neg1f, neg1i)


BN = 1000
BE = 2000
BE2 = 4000
BN2 = 2000


def _dot(x, w):
    return jnp.dot(x, w, preferred_element_type=F32)


def _t1_call(pos2, x2, h, enc_W, enc_b, w1a, w1b, b1):
    def body(pos_r, x_r, h_r, ew_r, eb_r, w1a_r, w1b_r, b1_r, z_r, m_r):
        z = jnp.maximum(
            pos_r[...] * ew_r[0:1, :] + x_r[...] * ew_r[1:2, :] + eb_r[...],
            0.0)
        z_r[...] = z
        q = _dot(z, w1a_r[...]) + _dot(h_r[...], w1b_r[...])
        m_r[...] = jnp.maximum(q + b1_r[...], 0.0)

    full = lambda shape: pl.BlockSpec(shape, lambda i: (0, 0))
    return pl.pallas_call(
        body,
        grid=(N // BN,),
        in_specs=[
            pl.BlockSpec((BN, 1), lambda i: (i, 0)),
            pl.BlockSpec((BN, 1), lambda i: (i, 0)),
            pl.BlockSpec((BN, D), lambda i: (i, 0)),
            full((2, D)), full((1, D)), full((D, D)), full((D, D)),
            full((1, D)),
        ],
        out_specs=[pl.BlockSpec((BN, D), lambda i: (i, 0))] * 2,
        out_shape=[jax.ShapeDtypeStruct((N, D), F32)] * 2,
    )(pos2, x2, h, enc_W, enc_b, w1a, w1b, b1)


def _t2_call(z, h, a0, a1, w2a, w2b, b2, dw1a, dw1b):
    def body(z_r, h_r, a0_r, a1_r, w2a_r, w2b_r, b2_r, dw1a_r, dw1b_r,
             hn_r, A_r, B_r):
        acc = _dot(z_r[...], w2a_r[...]) + _dot(h_r[...], w2b_r[...])
        hn = jnp.maximum(acc + b2_r[...] + a0_r[...] + a1_r[...], 0.0)
        hn_r[...] = hn
        A_r[...] = _dot(hn, dw1a_r[...])
        B_r[...] = _dot(hn, dw1b_r[...])

    full = lambda shape: pl.BlockSpec(shape, lambda i: (0, 0))
    blk = pl.BlockSpec((BN, D), lambda i: (i, 0))
    return pl.pallas_call(
        body,
        grid=(N // BN,),
        in_specs=[blk, blk, blk, blk,
                  full((D, D)), full((D, D)), full((1, D)),
                  full((D, D)), full((D, D))],
        out_specs=[blk] * 3,
        out_shape=[jax.ShapeDtypeStruct((N, D), F32)] * 3,
    )(z, h, a0, a1, w2a, w2b, b2, dw1a, dw1b)


def _t3_call(ga, gb, db1, dw2, db2):
    def body(ga_r, gb_r, db1_r, dw2_r, db2_r, out_r):
        x = jnp.maximum(ga_r[...] + gb_r[...] + db1_r[...], 0.0)
        u = _dot(x, dw2_r[...]) + db2_r[...]
        out_r[...] = jax.nn.sigmoid(u)

    full = lambda shape: pl.BlockSpec(shape, lambda i: (0, 0))
    blk = pl.BlockSpec((BE, D), lambda i: (i, 0))
    return pl.pallas_call(
        body,
        grid=(E // BE,),
        in_specs=[blk, blk, full((1, D)), full((D, 1)), full((1, 1))],
        out_specs=pl.BlockSpec((BE, 1), lambda i: (i, 0)),
        out_shape=jax.ShapeDtypeStruct((E, 1), F32),
    )(ga, gb, db1, dw2, db2)


def _t4_call(cnt):
    def body(cnt_r, out_r):
        ssum = jnp.sum(cnt_r[...], axis=0, keepdims=True)
        out_r[...] = jnp.where(ssum > 0.0, 1.0, 0.0).astype(F32)

    return pl.pallas_call(
        body,
        out_shape=jax.ShapeDtypeStruct((1, N), F32),
    )(cnt)


def _t5e_call(a1, a2, a3, edges_r, h1, h2, h3):
    eps = 1e-7

    def body(a1_r, a2_r, a3_r, e_r, h1_r, h2_r, h3_r, acc_r):
        e = e_r[...]

        def bce_sum(a_blk, t_blk):
            p = jnp.clip(a_blk, eps, 1.0 - eps)
            return jnp.sum(-(t_blk * jnp.log(p)
                             + (1.0 - t_blk) * jnp.log(1.0 - p)))

        lx = bce_sum(a3_r[...], e)
        lh = (bce_sum(a1_r[...], h1_r[...])
              + bce_sum(a2_r[...], h2_r[...])
              + bce_sum(a3_r[...], h3_r[...]))

        def cnt_match(a_blk):
            pred = jnp.where(a_blk > 0.4, 1.0, 0.0).astype(F32)
            return jnp.sum(jnp.where(pred == e, 1.0, 0.0))

        ce = cnt_match(a1_r[...]) + cnt_match(a2_r[...]) + cnt_match(a3_r[...])
        lanei = lax.broadcasted_iota(I32, (1, 128), 1)
        vec = (jnp.where(lanei == 0, lx, 0.0)
               + jnp.where(lanei == 1, lh, 0.0)
               + jnp.where(lanei == 2, ce, 0.0))
        acc_r[...] = vec

    return pl.pallas_call(
        body,
        out_shape=jax.ShapeDtypeStruct((1, 128), F32),
    )(a1, a2, a3, edges_r, h1, h2, h3)


def _t5n_call(bp, cp, pi_r, reach_r, rh_r):
    def body(bp_r, cp_r, pi_rr, re_r, rh_rr, acc_r):
        bpv = bp_r[...]
        best = jnp.max(bpv, axis=0, keepdims=True)
        cand = jnp.max(jnp.where(bpv == best, cp_r[...], -1),
                       axis=0, keepdims=True)
        col = lax.broadcasted_iota(I32, (1, N), 1)
        parents = jnp.where(cand >= 0, cand, col)
        cpar = jnp.sum(jnp.where(parents == pi_rr[...], 1.0, 0.0))
        crch = jnp.sum(jnp.where(jnp.round(re_r[...]) == rh_rr[...], 1.0, 0.0))
        lanei = lax.broadcasted_iota(I32, (1, 128), 1)
        vec = (jnp.where(lanei == 0, cpar, 0.0)
               + jnp.where(lanei == 1, crch, 0.0))
        acc_r[...] = vec

    return pl.pallas_call(
        body,
        out_shape=jax.ShapeDtypeStruct((1, 128), F32),
    )(bp, cp, pi_r, reach_r, rh_r)


def kernel(pos, s, edges, edges_h, reach_h, edge_index, pi,
           enc_W, enc_b, mp_W1, mp_b1, mp_W2, mp_b2,
           dec_W1, dec_b1, dec_W2, dec_b2):
    src = edge_index[0]
    dst = edge_index[1]
    hints = edges_h[1:]
    T = hints.shape[0]

    w1a, w1b = mp_W1[:D], mp_W1[D:]
    w2a, w2b = mp_W2[:D], mp_W2[D:]
    dw1a, dw1b = dec_W1[:D], dec_W1[D:]
    enc_b_r = enc_b.reshape(1, D)
    b1_r = mp_b1.reshape(1, D)
    b2_r = mp_b2.reshape(1, D)
    db1_r = dec_b1.reshape(1, D)
    db2_r = dec_b2.reshape(1, 1)

    pos2 = pos.reshape(N, 1).astype(F32)
    x2 = s.reshape(N, 1).astype(F32)
    zeros_nd = jnp.zeros((N, D), F32)
    zeros_n = jnp.zeros((N,), F32)
    neg1f = jnp.full((N,), -1.0, F32)
    neg1i = jnp.full((N,), -1, I32)

    h = zeros_nd
    alphas = []
    reach_row = None
    for _ in range(T):
        z, m = _t1_call(pos2, x2, h, enc_W, enc_b_r, w1a, w1b, b1_r)
        aggp = _k1_call(m, src, dst, zeros_nd)
        h, a_mat, b_mat = _t2_call(z, h, aggp[0], aggp[1],
                                   w2a, w2b, b2_r, dw1a, dw1b)
        ga, gb = _k2_call(a_mat, b_mat, src, dst)
        alpha = _t3_call(ga, gb, db1_r, dec_W2, db2_r).reshape(E)
        alphas.append(alpha)
        cntp = _k3_call(alpha, src, dst, zeros_n)
        reach_row = _t4_call(cntp.reshape(NW, N))
        x2 = reach_row.reshape(N, 1)

    bp, cp = _k4_call(alphas[-1], src, dst, neg1f, neg1i)
    bp = bp.reshape(NW, N)
    cp = cp.reshape(NW, N)
    acc_e = _t5e_call(alphas[0].reshape(1, E), alphas[1].reshape(1, E),
                      alphas[2].reshape(1, E), edges.reshape(1, E),
                      hints[0].reshape(1, E), hints[1].reshape(1, E),
                      hints[2].reshape(1, E))
    acc_n = _t5n_call(bp, cp, pi.reshape(1, N), reach_row,
                      reach_h[-1].reshape(1, N))

    loss_x = acc_e[0, 0] / E
    loss_h = acc_e[0, 1] / E
    edges_err = acc_e[0, 2] / (E * T) * 100.0
    reach_err = acc_n[0, 1] / N * 100.0
    parents_err = acc_n[0, 0] / N
    return jnp.stack([loss_x, loss_h, edges_err, reach_err, parents_err])


# speedup vs baseline: 5.2222x; 1.1779x over previous
import functools

import jax
import jax.numpy as jnp
from jax import lax
from jax.experimental import pallas as pl
from jax.experimental.pallas import tpu as pltpu
from jax.experimental.pallas import tpu_sc as plsc

F32 = jnp.float32
I32 = jnp.int32

N = 10000
E = 160000
D = 128

NC = 2
NS = 16
NW = NC * NS
CH = 128

E_HI = 5008
E_LO = 4992
NFULL = 39
HI_BASE_END = 16 * E_HI

def _chunk_start(wid):
    return jnp.where(wid < 2, wid * 40, 80 + (wid - 2) * 39)


def _wid_base():
    c = lax.axis_index("c")
    s = lax.axis_index("s")
    wid = s * NC + c
    base = jnp.where(wid < 16, wid * E_HI, HI_BASE_END + (wid - 16) * E_LO)
    return c, s, wid, base


def _sc_mesh():
    return plsc.VectorSubcoreMesh(core_axis_name="c", subcore_axis_name="s")


def _k1_call(m, src, dst, zeros_nd):
    @functools.partial(
        pl.kernel,
        out_type=jax.ShapeDtypeStruct((NC, N, D), F32),
        mesh=_sc_mesh(),
        compiler_params=pltpu.CompilerParams(needs_layout_passes=False),
        scratch_types=[
            pltpu.VMEM(((NFULL + 1) * CH,), I32),
            pltpu.VMEM((CH,), I32),
            pltpu.VMEM((CH,), I32),
            pltpu.VMEM((2, CH, D), F32),
            pltpu.VMEM_SHARED((N, D), F32),
            pltpu.SemaphoreType.DMA,
            pltpu.SemaphoreType.DMA,
            pltpu.SemaphoreType.DMA,
            pltpu.SemaphoreType.DMA,
        ],
    )
    def k1(m_hbm, src_hbm, dst_hbm, z_hbm, out_hbm,
           sidx_all, didx0, didx1, rows, acc, sg0, sg1, sa0, sa1):
        c, s, wid, base = _wid_base()
        didx = (didx0, didx1)
        sg = (sg0, sg1)
        sa = (sa0, sa1)

        @pl.when(s < 15)
        def _():
            pltpu.sync_copy(z_hbm.at[pl.ds(s * 632, 632)],
                            acc.at[pl.ds(s * 632, 632)])

        @pl.when(s == 15)
        def _():
            pltpu.sync_copy(z_hbm.at[pl.ds(9480, 520)],
                            acc.at[pl.ds(9480, 520)])

        plsc.subcore_barrier()

        ebase = _chunk_start(wid) * CH
        nf = NFULL * CH
        pltpu.sync_copy(src_hbm.at[pl.ds(ebase, nf)],
                        sidx_all.at[pl.ds(0, nf)])

        @pl.when(wid < 2)
        def _():
            pltpu.sync_copy(src_hbm.at[pl.ds(ebase + nf, CH)],
                            sidx_all.at[pl.ds(nf, CH)])

        gather_h = [None, None]
        add_h = [None, None]

        def issue_gather(k, sl):
            isl = sidx_all.at[pl.ds(k * CH, CH)]
            gather_h[sl] = pltpu.async_copy(m_hbm.at[isl], rows.at[sl],
                                            sg[sl])

        issue_gather(0, 0)
        pltpu.sync_copy(dst_hbm.at[pl.ds(ebase, CH)], didx[0])
        for k in range(NFULL):
            sl = k % 2
            ns = 1 - sl
            if k + 1 < NFULL:
                if add_h[ns] is not None:
                    add_h[ns].wait()
                issue_gather(k + 1, ns)
                pltpu.sync_copy(dst_hbm.at[pl.ds(ebase + (k + 1) * CH, CH)],
                                didx[ns])
            gather_h[sl].wait()
            add_h[sl] = pltpu.async_copy(
                rows.at[sl], acc.at[didx[sl]], sa[sl], add=True)
        for hnd in add_h:
            if hnd is not None:
                hnd.wait()

        @pl.when(wid < 2)
        def _():
            off = ebase + nf
            issue_gather(NFULL, 0)
            pltpu.sync_copy(dst_hbm.at[pl.ds(off, CH)], didx0)
            gather_h[0].wait()
            pltpu.async_copy(rows.at[0], acc.at[didx0], sa0, add=True).wait()

        plsc.subcore_barrier()

        @pl.when(s < 15)
        def _():
            pltpu.sync_copy(acc.at[pl.ds(s * 632, 632)],
                            out_hbm.at[c, pl.ds(s * 632, 632)])

        @pl.when(s == 15)
        def _():
            pltpu.sync_copy(acc.at[pl.ds(9480, 520)],
                            out_hbm.at[c, pl.ds(9480, 520)])

    return k1(m, src, dst, zeros_nd)


def _k2_call(a, b, src, dst):
    @functools.partial(
        pl.kernel,
        out_type=(jax.ShapeDtypeStruct((E, D), F32),
                  jax.ShapeDtypeStruct((E, D), F32)),
        mesh=_sc_mesh(),
        compiler_params=pltpu.CompilerParams(needs_layout_passes=False),
        scratch_types=[
            pltpu.VMEM(((NFULL + 1) * CH,), I32),
            pltpu.VMEM(((NFULL + 1) * CH,), I32),
            pltpu.VMEM((2, CH, D), F32),
            pltpu.VMEM((2, CH, D), F32),
            pltpu.SemaphoreType.DMA,
            pltpu.SemaphoreType.DMA,
            pltpu.SemaphoreType.DMA,
            pltpu.SemaphoreType.DMA,
            pltpu.SemaphoreType.DMA,
            pltpu.SemaphoreType.DMA,
            pltpu.SemaphoreType.DMA,
            pltpu.SemaphoreType.DMA,
        ],
    )
    def k2(a_hbm, b_hbm, src_hbm, dst_hbm, ga_hbm, gb_hbm,
           sidx_all, didx_all, rowsa, rowsb,
           sga0, sga1, sgb0, sgb1, swa0, swa1, swb0, swb1):
        c, s, wid, base = _wid_base()
        sga = (sga0, sga1)
        sgb = (sgb0, sgb1)
        swa = (swa0, swa1)
        swb = (swb0, swb1)
        ebase = _chunk_start(wid) * CH
        nf = NFULL * CH
        pltpu.sync_copy(src_hbm.at[pl.ds(ebase, nf)],
                        sidx_all.at[pl.ds(0, nf)])
        pltpu.sync_copy(dst_hbm.at[pl.ds(ebase, nf)],
                        didx_all.at[pl.ds(0, nf)])

        @pl.when(wid < 2)
        def _():
            pltpu.sync_copy(src_hbm.at[pl.ds(ebase + nf, CH)],
                            sidx_all.at[pl.ds(nf, CH)])
            pltpu.sync_copy(dst_hbm.at[pl.ds(ebase + nf, CH)],
                            didx_all.at[pl.ds(nf, CH)])

        ga_h = [None, None]
        gb_h = [None, None]
        wa_h = [None, None]
        wb_h = [None, None]

        def issue_gathers(k, sl):
            isl = sidx_all.at[pl.ds(k * CH, CH)]
            idl = didx_all.at[pl.ds(k * CH, CH)]
            ga_h[sl] = pltpu.async_copy(a_hbm.at[isl], rowsa.at[sl], sga[sl])
            gb_h[sl] = pltpu.async_copy(b_hbm.at[idl], rowsb.at[sl], sgb[sl])

        issue_gathers(0, 0)
        for k in range(NFULL):
            sl = k % 2
            ns = 1 - sl
            if k + 1 < NFULL:
                if wa_h[ns] is not None:
                    wa_h[ns].wait()
                    wb_h[ns].wait()
                issue_gathers(k + 1, ns)
            ga_h[sl].wait()
            gb_h[sl].wait()
            off = ebase + k * CH
            wa_h[sl] = pltpu.async_copy(rowsa.at[sl],
                                        ga_hbm.at[pl.ds(off, CH)], swa[sl])
            wb_h[sl] = pltpu.async_copy(rowsb.at[sl],
                                        gb_hbm.at[pl.ds(off, CH)], swb[sl])
        for hnd in wa_h + wb_h:
            if hnd is not None:
                hnd.wait()

        @pl.when(wid < 2)
        def _():
            off = ebase + nf
            issue_gathers(NFULL, 0)
            ga_h[0].wait()
            gb_h[0].wait()
            pltpu.sync_copy(rowsa.at[0], ga_hbm.at[pl.ds(off, CH)])
            pltpu.sync_copy(rowsb.at[0], gb_hbm.at[pl.ds(off, CH)])

    return k2(a, b, src, dst)


def _k3_call(alpha, src, dst, zeros_n):
    @functools.partial(
        pl.kernel,
        out_type=jax.ShapeDtypeStruct((NW, 1, N), F32),
        mesh=_sc_mesh(),
        compiler_params=pltpu.CompilerParams(needs_layout_passes=False),
        scratch_types=[
            pltpu.VMEM((E_HI,), F32),
            pltpu.VMEM((E_HI,), I32),
            pltpu.VMEM((E_HI,), I32),
            pltpu.VMEM((N,), F32),
        ],
    )
    def k3(a_hbm, src_hbm, dst_hbm, z_hbm, out_hbm, aall, sall, dall, cnt):
        c, s, wid, base = _wid_base()
        pltpu.sync_copy(z_hbm, cnt)
        pltpu.sync_copy(a_hbm.at[pl.ds(base, E_LO)], aall.at[pl.ds(0, E_LO)])
        pltpu.sync_copy(src_hbm.at[pl.ds(base, E_LO)],
                        sall.at[pl.ds(0, E_LO)])
        pltpu.sync_copy(dst_hbm.at[pl.ds(base, E_LO)],
                        dall.at[pl.ds(0, E_LO)])

        @pl.when(wid < 16)
        def _():
            pltpu.sync_copy(a_hbm.at[pl.ds(base + E_LO, 16)],
                            aall.at[pl.ds(E_LO, 16)])
            pltpu.sync_copy(src_hbm.at[pl.ds(base + E_LO, 16)],
                            sall.at[pl.ds(E_LO, 16)])
            pltpu.sync_copy(dst_hbm.at[pl.ds(base + E_LO, 16)],
                            dall.at[pl.ds(E_LO, 16)])

        ones = jnp.full((16,), 1.0, F32)

        def group(g):
            a16 = aall[pl.ds(g * 16, 16)]
            m16 = a16 >= 0.4
            plsc.store_scatter(cnt, [sall[pl.ds(g * 16, 16)]], ones,
                               mask=m16)
            plsc.store_scatter(cnt, [dall[pl.ds(g * 16, 16)]], ones,
                               mask=m16)

        for g in range(E_LO // 16):
            group(g)

        @pl.when(wid < 16)
        def _():
            group(E_LO // 16)

        pltpu.sync_copy(cnt, out_hbm.at[wid, 0])

    return k3(alpha, src, dst, zeros_n)


def _k4_call(alpha, src, dst, neg1f, neg1i):
    @functools.partial(
        pl.kernel,
        out_type=(jax.ShapeDtypeStruct((NW, 1, N), F32),
                  jax.ShapeDtypeStruct((NW, 1, N), I32)),
        mesh=_sc_mesh(),
        compiler_params=pltpu.CompilerParams(needs_layout_passes=False),
        scratch_types=[
            pltpu.VMEM((CH,), F32),
            pltpu.VMEM((CH,), I32),
            pltpu.VMEM((CH,), I32),
            pltpu.VMEM((16,), F32),
            pltpu.VMEM((16,), I32),
            pltpu.VMEM((16,), I32),
            pltpu.VMEM((N,), F32),
            pltpu.VMEM((N,), I32),
        ],
    )
    def k4(a_hbm, src_hbm, dst_hbm, nf_hbm, ni_hbm, bout_hbm, cout_hbm,
           av, sidx, didx, av16, sidx16, didx16, best, cand):
        c, s, wid, base = _wid_base()
        pltpu.sync_copy(nf_hbm, best)
        pltpu.sync_copy(ni_hbm, cand)
        lane = jnp.arange(16, dtype=I32)

        def groups(avr, sr, dr, ng):
            for g in range(ng):
                a16 = avr[pl.ds(g * 16, 16)]
                s16 = sr[pl.ds(g * 16, 16)]
                d16 = dr[pl.ds(g * 16, 16)]
                for j in range(16):
                    m = lane == j
                    b16 = plsc.load_gather(best, [d16])
                    c16 = plsc.load_gather(cand, [d16])
                    gt = a16 > b16
                    eq = a16 == b16
                    nb = jnp.where(gt, a16, b16)
                    ncd = jnp.where(gt, s16,
                                    jnp.where(eq, jnp.maximum(c16, s16), c16))
                    plsc.store_scatter(best, [d16], nb, mask=m)
                    plsc.store_scatter(cand, [d16], ncd, mask=m)

        def chunk(k, carry):
            off = base + k * CH
            pltpu.sync_copy(a_hbm.at[pl.ds(off, CH)], av)
            pltpu.sync_copy(src_hbm.at[pl.ds(off, CH)], sidx)
            pltpu.sync_copy(dst_hbm.at[pl.ds(off, CH)], didx)
            groups(av, sidx, didx, CH // 16)
            return carry

        lax.fori_loop(0, NFULL, chunk, 0)

        @pl.when(wid < 16)
        def _():
            off = base + NFULL * CH
            pltpu.sync_copy(a_hbm.at[pl.ds(off, 16)], av16)
            pltpu.sync_copy(src_hbm.at[pl.ds(off, 16)], sidx16)
            pltpu.sync_copy(dst_hbm.at[pl.ds(off, 16)], didx16)
            groups(av16, sidx16, didx16, 1)

        pltpu.sync_copy(best, bout_hbm.at[wid, 0])
        pltpu.sync_copy(cand, cout_hbm.at[wid, 0])

    return k4(alpha, src, dst, neg1f, neg1i)


BN = 1000
BE = 2000
BE2 = 4000
BN2 = 2000


def _dot(x, w):
    return jnp.dot(x, w, preferred_element_type=F32)


def _t1_call(pos2, x2, h, enc_W, enc_b, w1a, w1b, b1):
    def body(pos_r, x_r, h_r, ew_r, eb_r, w1a_r, w1b_r, b1_r, z_r, m_r):
        z = jnp.maximum(
            pos_r[...] * ew_r[0:1, :] + x_r[...] * ew_r[1:2, :] + eb_r[...],
            0.0)
        z_r[...] = z
        q = _dot(z, w1a_r[...]) + _dot(h_r[...], w1b_r[...])
        m_r[...] = jnp.maximum(q + b1_r[...], 0.0)

    full = lambda shape: pl.BlockSpec(shape, lambda i: (0, 0))
    return pl.pallas_call(
        body,
        grid=(N // BN,),
        in_specs=[
            pl.BlockSpec((BN, 1), lambda i: (i, 0)),
            pl.BlockSpec((BN, 1), lambda i: (i, 0)),
            pl.BlockSpec((BN, D), lambda i: (i, 0)),
            full((2, D)), full((1, D)), full((D, D)), full((D, D)),
            full((1, D)),
        ],
        out_specs=[pl.BlockSpec((BN, D), lambda i: (i, 0))] * 2,
        out_shape=[jax.ShapeDtypeStruct((N, D), F32)] * 2,
    )(pos2, x2, h, enc_W, enc_b, w1a, w1b, b1)


def _t2_call(z, h, a0, a1, w2a, w2b, b2, dw1a, dw1b):
    def body(z_r, h_r, a0_r, a1_r, w2a_r, w2b_r, b2_r, dw1a_r, dw1b_r,
             hn_r, A_r, B_r):
        acc = _dot(z_r[...], w2a_r[...]) + _dot(h_r[...], w2b_r[...])
        hn = jnp.maximum(acc + b2_r[...] + a0_r[...] + a1_r[...], 0.0)
        hn_r[...] = hn
        A_r[...] = _dot(hn, dw1a_r[...])
        B_r[...] = _dot(hn, dw1b_r[...])

    full = lambda shape: pl.BlockSpec(shape, lambda i: (0, 0))
    blk = pl.BlockSpec((BN, D), lambda i: (i, 0))
    return pl.pallas_call(
        body,
        grid=(N // BN,),
        in_specs=[blk, blk, blk, blk,
                  full((D, D)), full((D, D)), full((1, D)),
                  full((D, D)), full((D, D))],
        out_specs=[blk] * 3,
        out_shape=[jax.ShapeDtypeStruct((N, D), F32)] * 3,
    )(z, h, a0, a1, w2a, w2b, b2, dw1a, dw1b)


def _t3_call(ga, gb, db1, dw2, db2):
    def body(ga_r, gb_r, db1_r, dw2_r, db2_r, out_r):
        x = jnp.maximum(ga_r[...] + gb_r[...] + db1_r[...], 0.0)
        u = _dot(x, dw2_r[...]) + db2_r[...]
        out_r[...] = jax.nn.sigmoid(u)

    full = lambda shape: pl.BlockSpec(shape, lambda i: (0, 0))
    blk = pl.BlockSpec((BE, D), lambda i: (i, 0))
    return pl.pallas_call(
        body,
        grid=(E // BE,),
        in_specs=[blk, blk, full((1, D)), full((D, 1)), full((1, 1))],
        out_specs=pl.BlockSpec((BE, 1), lambda i: (i, 0)),
        out_shape=jax.ShapeDtypeStruct((E, 1), F32),
    )(ga, gb, db1, dw2, db2)


def _t4_call(cnt):
    def body(cnt_r, out_r):
        ssum = jnp.sum(cnt_r[...], axis=0, keepdims=True)
        out_r[...] = jnp.where(ssum > 0.0, 1.0, 0.0).astype(F32)

    return pl.pallas_call(
        body,
        out_shape=jax.ShapeDtypeStruct((1, N), F32),
    )(cnt)


def _t5e_call(a1, a2, a3, edges_r, h1, h2, h3):
    eps = 1e-7

    def body(a1_r, a2_r, a3_r, e_r, h1_r, h2_r, h3_r, acc_r):
        e = e_r[...]

        def bce_sum(a_blk, t_blk):
            p = jnp.clip(a_blk, eps, 1.0 - eps)
            return jnp.sum(-(t_blk * jnp.log(p)
                             + (1.0 - t_blk) * jnp.log(1.0 - p)))

        lx = bce_sum(a3_r[...], e)
        lh = (bce_sum(a1_r[...], h1_r[...])
              + bce_sum(a2_r[...], h2_r[...])
              + bce_sum(a3_r[...], h3_r[...]))

        def cnt_match(a_blk):
            pred = jnp.where(a_blk > 0.4, 1.0, 0.0).astype(F32)
            return jnp.sum(jnp.where(pred == e, 1.0, 0.0))

        ce = cnt_match(a1_r[...]) + cnt_match(a2_r[...]) + cnt_match(a3_r[...])
        lanei = lax.broadcasted_iota(I32, (1, 128), 1)
        vec = (jnp.where(lanei == 0, lx, 0.0)
               + jnp.where(lanei == 1, lh, 0.0)
               + jnp.where(lanei == 2, ce, 0.0))
        acc_r[...] = vec

    return pl.pallas_call(
        body,
        out_shape=jax.ShapeDtypeStruct((1, 128), F32),
    )(a1, a2, a3, edges_r, h1, h2, h3)


def _t5n_call(bp, cp, pi_r, reach_r, rh_r):
    def body(bp_r, cp_r, pi_rr, re_r, rh_rr, acc_r):
        bpv = bp_r[...]
        best = jnp.max(bpv, axis=0, keepdims=True)
        cand = jnp.max(jnp.where(bpv == best, cp_r[...], -1),
                       axis=0, keepdims=True)
        col = lax.broadcasted_iota(I32, (1, N), 1)
        parents = jnp.where(cand >= 0, cand, col)
        cpar = jnp.sum(jnp.where(parents == pi_rr[...], 1.0, 0.0))
        crch = jnp.sum(jnp.where(jnp.round(re_r[...]) == rh_rr[...], 1.0, 0.0))
        lanei = lax.broadcasted_iota(I32, (1, 128), 1)
        vec = (jnp.where(lanei == 0, cpar, 0.0)
               + jnp.where(lanei == 1, crch, 0.0))
        acc_r[...] = vec

    return pl.pallas_call(
        body,
        out_shape=jax.ShapeDtypeStruct((1, 128), F32),
    )(bp, cp, pi_r, reach_r, rh_r)


def kernel(pos, s, edges, edges_h, reach_h, edge_index, pi,
           enc_W, enc_b, mp_W1, mp_b1, mp_W2, mp_b2,
           dec_W1, dec_b1, dec_W2, dec_b2):
    src = edge_index[0]
    dst = edge_index[1]
    hints = edges_h[1:]
    T = hints.shape[0]

    w1a, w1b = mp_W1[:D], mp_W1[D:]
    w2a, w2b = mp_W2[:D], mp_W2[D:]
    dw1a, dw1b = dec_W1[:D], dec_W1[D:]
    enc_b_r = enc_b.reshape(1, D)
    b1_r = mp_b1.reshape(1, D)
    b2_r = mp_b2.reshape(1, D)
    db1_r = dec_b1.reshape(1, D)
    db2_r = dec_b2.reshape(1, 1)

    pos2 = pos.reshape(N, 1).astype(F32)
    x2 = s.reshape(N, 1).astype(F32)
    zeros_nd = jnp.zeros((N, D), F32)
    zeros_n = jnp.zeros((N,), F32)
    neg1f = jnp.full((N,), -1.0, F32)
    neg1i = jnp.full((N,), -1, I32)

    h = zeros_nd
    alphas = []
    reach_row = None
    for _ in range(T):
        z, m = _t1_call(pos2, x2, h, enc_W, enc_b_r, w1a, w1b, b1_r)
        aggp = _k1_call(m, src, dst, zeros_nd)
        h, a_mat, b_mat = _t2_call(z, h, aggp[0], aggp[1],
                                   w2a, w2b, b2_r, dw1a, dw1b)
        ga, gb = _k2_call(a_mat, b_mat, src, dst)
        alpha = _t3_call(ga, gb, db1_r, dec_W2, db2_r).reshape(E)
        alphas.append(alpha)
        cntp = _k3_call(alpha, src, dst, zeros_n)
        reach_row = _t4_call(cntp.reshape(NW, N))
        x2 = reach_row.reshape(N, 1)

    bp, cp = _k4_call(alphas[-1], src, dst, neg1f, neg1i)
    bp = bp.reshape(NW, N)
    cp = cp.reshape(NW, N)
    acc_e = _t5e_call(alphas[0].reshape(1, E), alphas[1].reshape(1, E),
                      alphas[2].reshape(1, E), edges.reshape(1, E),
                      hints[0].reshape(1, E), hints[1].reshape(1, E),
                      hints[2].reshape(1, E))
    acc_n = _t5n_call(bp, cp, pi.reshape(1, N), reach_row,
                      reach_h[-1].reshape(1, N))

    loss_x = acc_e[0, 0] / E
    loss_h = acc_e[0, 1] / E
    edges_err = acc_e[0, 2] / (E * T) * 100.0
    reach_err = acc_n[0, 1] / N * 100.0
    parents_err = acc_n[0, 0] / N
    return jnp.stack([loss_x, loss_h, edges_err, reach_err, parents_err])


# speedup vs baseline: 5.4431x; 1.0423x over previous
import functools

import jax
import jax.numpy as jnp
from jax import lax
from jax.experimental import pallas as pl
from jax.experimental.pallas import tpu as pltpu
from jax.experimental.pallas import tpu_sc as plsc

F32 = jnp.float32
I32 = jnp.int32

N = 10000
E = 160000
D = 128

NC = 2
NS = 16
NW = NC * NS
CH = 128

E_HI = 5008
E_LO = 4992
NFULL = 39
HI_BASE_END = 16 * E_HI

def _chunk_start(wid):
    return jnp.where(wid < 2, wid * 40, 80 + (wid - 2) * 39)


def _wid_base():
    c = lax.axis_index("c")
    s = lax.axis_index("s")
    wid = s * NC + c
    base = jnp.where(wid < 16, wid * E_HI, HI_BASE_END + (wid - 16) * E_LO)
    return c, s, wid, base


def _sc_mesh():
    return plsc.VectorSubcoreMesh(core_axis_name="c", subcore_axis_name="s")


def _k1_call(m, src, dst, zeros_nd):
    @functools.partial(
        pl.kernel,
        out_type=jax.ShapeDtypeStruct((NC, N, D), F32),
        mesh=_sc_mesh(),
        compiler_params=pltpu.CompilerParams(needs_layout_passes=False),
        scratch_types=[
            pltpu.VMEM(((NFULL + 1) * CH,), I32),
            pltpu.VMEM((CH,), I32),
            pltpu.VMEM((CH,), I32),
            pltpu.VMEM((2, CH, D), F32),
            pltpu.VMEM_SHARED((N, D), F32),
            pltpu.SemaphoreType.DMA,
            pltpu.SemaphoreType.DMA,
            pltpu.SemaphoreType.DMA,
            pltpu.SemaphoreType.DMA,
        ],
    )
    def k1(m_hbm, src_hbm, dst_hbm, z_hbm, out_hbm,
           sidx_all, didx0, didx1, rows, acc, sg0, sg1, sa0, sa1):
        c, s, wid, base = _wid_base()
        didx = (didx0, didx1)
        sg = (sg0, sg1)
        sa = (sa0, sa1)

        @pl.when(s < 15)
        def _():
            pltpu.sync_copy(z_hbm.at[pl.ds(s * 632, 632)],
                            acc.at[pl.ds(s * 632, 632)])

        @pl.when(s == 15)
        def _():
            pltpu.sync_copy(z_hbm.at[pl.ds(9480, 520)],
                            acc.at[pl.ds(9480, 520)])

        plsc.subcore_barrier()

        ebase = _chunk_start(wid) * CH
        nf = NFULL * CH
        pltpu.sync_copy(src_hbm.at[pl.ds(ebase, nf)],
                        sidx_all.at[pl.ds(0, nf)])

        @pl.when(wid < 2)
        def _():
            pltpu.sync_copy(src_hbm.at[pl.ds(ebase + nf, CH)],
                            sidx_all.at[pl.ds(nf, CH)])

        gather_h = [None, None]
        add_h = [None, None]

        def issue_gather(k, sl):
            isl = sidx_all.at[pl.ds(k * CH, CH)]
            gather_h[sl] = pltpu.async_copy(m_hbm.at[isl], rows.at[sl],
                                            sg[sl])

        issue_gather(0, 0)
        pltpu.sync_copy(dst_hbm.at[pl.ds(ebase, CH)], didx[0])
        for k in range(NFULL):
            sl = k % 2
            ns = 1 - sl
            if k + 1 < NFULL:
                if add_h[ns] is not None:
                    add_h[ns].wait()
                issue_gather(k + 1, ns)
                pltpu.sync_copy(dst_hbm.at[pl.ds(ebase + (k + 1) * CH, CH)],
                                didx[ns])
            gather_h[sl].wait()
            add_h[sl] = pltpu.async_copy(
                rows.at[sl], acc.at[didx[sl]], sa[sl], add=True)
        for hnd in add_h:
            if hnd is not None:
                hnd.wait()

        @pl.when(wid < 2)
        def _():
            off = ebase + nf
            issue_gather(NFULL, 0)
            pltpu.sync_copy(dst_hbm.at[pl.ds(off, CH)], didx0)
            gather_h[0].wait()
            pltpu.async_copy(rows.at[0], acc.at[didx0], sa0, add=True).wait()

        plsc.subcore_barrier()

        @pl.when(s < 15)
        def _():
            pltpu.sync_copy(acc.at[pl.ds(s * 632, 632)],
                            out_hbm.at[c, pl.ds(s * 632, 632)])

        @pl.when(s == 15)
        def _():
            pltpu.sync_copy(acc.at[pl.ds(9480, 520)],
                            out_hbm.at[c, pl.ds(9480, 520)])

    return k1(m, src, dst, zeros_nd)


def _k2_call(a, b, src, dst):
    @functools.partial(
        pl.kernel,
        out_type=(jax.ShapeDtypeStruct((E, D), F32),
                  jax.ShapeDtypeStruct((E, D), F32)),
        mesh=_sc_mesh(),
        compiler_params=pltpu.CompilerParams(needs_layout_passes=False),
        scratch_types=[
            pltpu.VMEM(((NFULL + 1) * CH,), I32),
            pltpu.VMEM(((NFULL + 1) * CH,), I32),
            pltpu.VMEM((2, CH, D), F32),
            pltpu.VMEM((2, CH, D), F32),
            pltpu.SemaphoreType.DMA,
            pltpu.SemaphoreType.DMA,
            pltpu.SemaphoreType.DMA,
            pltpu.SemaphoreType.DMA,
            pltpu.SemaphoreType.DMA,
            pltpu.SemaphoreType.DMA,
            pltpu.SemaphoreType.DMA,
            pltpu.SemaphoreType.DMA,
        ],
    )
    def k2(a_hbm, b_hbm, src_hbm, dst_hbm, ga_hbm, gb_hbm,
           sidx_all, didx_all, rowsa, rowsb,
           sga0, sga1, sgb0, sgb1, swa0, swa1, swb0, swb1):
        c, s, wid, base = _wid_base()
        sga = (sga0, sga1)
        sgb = (sgb0, sgb1)
        swa = (swa0, swa1)
        swb = (swb0, swb1)
        ebase = _chunk_start(wid) * CH
        nf = NFULL * CH
        pltpu.sync_copy(src_hbm.at[pl.ds(ebase, nf)],
                        sidx_all.at[pl.ds(0, nf)])
        pltpu.sync_copy(dst_hbm.at[pl.ds(ebase, nf)],
                        didx_all.at[pl.ds(0, nf)])

        @pl.when(wid < 2)
        def _():
            pltpu.sync_copy(src_hbm.at[pl.ds(ebase + nf, CH)],
                            sidx_all.at[pl.ds(nf, CH)])
            pltpu.sync_copy(dst_hbm.at[pl.ds(ebase + nf, CH)],
                            didx_all.at[pl.ds(nf, CH)])

        ga_h = [None, None]
        gb_h = [None, None]
        wa_h = [None, None]
        wb_h = [None, None]

        def issue_gathers(k, sl):
            isl = sidx_all.at[pl.ds(k * CH, CH)]
            idl = didx_all.at[pl.ds(k * CH, CH)]
            ga_h[sl] = pltpu.async_copy(a_hbm.at[isl], rowsa.at[sl], sga[sl])
            gb_h[sl] = pltpu.async_copy(b_hbm.at[idl], rowsb.at[sl], sgb[sl])

        issue_gathers(0, 0)
        for k in range(NFULL):
            sl = k % 2
            ns = 1 - sl
            if k + 1 < NFULL:
                if wa_h[ns] is not None:
                    wa_h[ns].wait()
                    wb_h[ns].wait()
                issue_gathers(k + 1, ns)
            ga_h[sl].wait()
            gb_h[sl].wait()
            off = ebase + k * CH
            wa_h[sl] = pltpu.async_copy(rowsa.at[sl],
                                        ga_hbm.at[pl.ds(off, CH)], swa[sl])
            wb_h[sl] = pltpu.async_copy(rowsb.at[sl],
                                        gb_hbm.at[pl.ds(off, CH)], swb[sl])
        for hnd in wa_h + wb_h:
            if hnd is not None:
                hnd.wait()

        @pl.when(wid < 2)
        def _():
            off = ebase + nf
            issue_gathers(NFULL, 0)
            ga_h[0].wait()
            gb_h[0].wait()
            pltpu.sync_copy(rowsa.at[0], ga_hbm.at[pl.ds(off, CH)])
            pltpu.sync_copy(rowsb.at[0], gb_hbm.at[pl.ds(off, CH)])

    return k2(a, b, src, dst)


def _k3_call(alpha, src, dst, zeros_n):
    @functools.partial(
        pl.kernel,
        out_type=jax.ShapeDtypeStruct((NW, 1, N), F32),
        mesh=_sc_mesh(),
        compiler_params=pltpu.CompilerParams(needs_layout_passes=False),
        scratch_types=[
            pltpu.VMEM((E_HI,), F32),
            pltpu.VMEM((E_HI,), I32),
            pltpu.VMEM((E_HI,), I32),
            pltpu.VMEM((N,), F32),
        ],
    )
    def k3(a_hbm, src_hbm, dst_hbm, z_hbm, out_hbm, aall, sall, dall, cnt):
        c, s, wid, base = _wid_base()
        pltpu.sync_copy(z_hbm, cnt)
        pltpu.sync_copy(a_hbm.at[pl.ds(base, E_LO)], aall.at[pl.ds(0, E_LO)])
        pltpu.sync_copy(src_hbm.at[pl.ds(base, E_LO)],
                        sall.at[pl.ds(0, E_LO)])
        pltpu.sync_copy(dst_hbm.at[pl.ds(base, E_LO)],
                        dall.at[pl.ds(0, E_LO)])

        @pl.when(wid < 16)
        def _():
            pltpu.sync_copy(a_hbm.at[pl.ds(base + E_LO, 16)],
                            aall.at[pl.ds(E_LO, 16)])
            pltpu.sync_copy(src_hbm.at[pl.ds(base + E_LO, 16)],
                            sall.at[pl.ds(E_LO, 16)])
            pltpu.sync_copy(dst_hbm.at[pl.ds(base + E_LO, 16)],
                            dall.at[pl.ds(E_LO, 16)])

        ones = jnp.full((16,), 1.0, F32)

        def group(g):
            a16 = aall[pl.ds(g * 16, 16)]
            m16 = a16 >= 0.4
            plsc.store_scatter(cnt, [sall[pl.ds(g * 16, 16)]], ones,
                               mask=m16)
            plsc.store_scatter(cnt, [dall[pl.ds(g * 16, 16)]], ones,
                               mask=m16)

        for g in range(E_LO // 16):
            group(g)

        @pl.when(wid < 16)
        def _():
            group(E_LO // 16)

        pltpu.sync_copy(cnt, out_hbm.at[wid, 0])

    return k3(alpha, src, dst, zeros_n)


def _k4_call(alpha, src, dst, neg1f, neg1i):
    @functools.partial(
        pl.kernel,
        out_type=(jax.ShapeDtypeStruct((NW, 1, N), F32),
                  jax.ShapeDtypeStruct((NW, 1, N), I32)),
        mesh=_sc_mesh(),
        compiler_params=pltpu.CompilerParams(needs_layout_passes=False),
        scratch_types=[
            pltpu.VMEM((E_HI,), F32),
            pltpu.VMEM((E_HI,), I32),
            pltpu.VMEM((E_HI,), I32),
            pltpu.VMEM((N,), F32),
            pltpu.VMEM((N,), I32),
        ],
    )
    def k4(a_hbm, src_hbm, dst_hbm, nf_hbm, ni_hbm, bout_hbm, cout_hbm,
           aall, sall, dall, best, cand):
        c, s, wid, base = _wid_base()
        pltpu.sync_copy(nf_hbm, best)
        pltpu.sync_copy(ni_hbm, cand)
        pltpu.sync_copy(a_hbm.at[pl.ds(base, E_LO)], aall.at[pl.ds(0, E_LO)])
        pltpu.sync_copy(src_hbm.at[pl.ds(base, E_LO)],
                        sall.at[pl.ds(0, E_LO)])
        pltpu.sync_copy(dst_hbm.at[pl.ds(base, E_LO)],
                        dall.at[pl.ds(0, E_LO)])

        @pl.when(wid < 16)
        def _():
            pltpu.sync_copy(a_hbm.at[pl.ds(base + E_LO, 16)],
                            aall.at[pl.ds(E_LO, 16)])
            pltpu.sync_copy(src_hbm.at[pl.ds(base + E_LO, 16)],
                            sall.at[pl.ds(E_LO, 16)])
            pltpu.sync_copy(dst_hbm.at[pl.ds(base + E_LO, 16)],
                            dall.at[pl.ds(E_LO, 16)])

        lane = jnp.arange(16, dtype=I32)

        def group(goff):
            a16 = aall[pl.ds(goff, 16)]
            s16 = sall[pl.ds(goff, 16)]
            d16 = dall[pl.ds(goff, 16)]
            for j in range(16):
                m = lane == j
                b16 = plsc.load_gather(best, [d16])
                c16 = plsc.load_gather(cand, [d16])
                gt = a16 > b16
                eq = a16 == b16
                nb = jnp.where(gt, a16, b16)
                ncd = jnp.where(gt, s16,
                                jnp.where(eq, jnp.maximum(c16, s16), c16))
                plsc.store_scatter(best, [d16], nb, mask=m)
                plsc.store_scatter(cand, [d16], ncd, mask=m)

        def body(g, carry):
            group(g * 16)
            return carry

        lax.fori_loop(0, E_LO // 16, body, 0)

        @pl.when(wid < 16)
        def _():
            group(E_LO)

        pltpu.sync_copy(best, bout_hbm.at[wid, 0])
        pltpu.sync_copy(cand, cout_hbm.at[wid, 0])

    return k4(alpha, src, dst, neg1f, neg1i)


BN = 1000
BE = 2000
BE2 = 4000
BN2 = 2000


def _dot(x, w):
    return jnp.dot(x, w, preferred_element_type=F32)


def _t1_call(pos2, x2, h, enc_W, enc_b, w1a, w1b, b1):
    def body(pos_r, x_r, h_r, ew_r, eb_r, w1a_r, w1b_r, b1_r, z_r, m_r):
        z = jnp.maximum(
            pos_r[...] * ew_r[0:1, :] + x_r[...] * ew_r[1:2, :] + eb_r[...],
            0.0)
        z_r[...] = z
        q = _dot(z, w1a_r[...]) + _dot(h_r[...], w1b_r[...])
        m_r[...] = jnp.maximum(q + b1_r[...], 0.0)

    full = lambda shape: pl.BlockSpec(shape, lambda i: (0, 0))
    return pl.pallas_call(
        body,
        grid=(N // BN,),
        in_specs=[
            pl.BlockSpec((BN, 1), lambda i: (i, 0)),
            pl.BlockSpec((BN, 1), lambda i: (i, 0)),
            pl.BlockSpec((BN, D), lambda i: (i, 0)),
            full((2, D)), full((1, D)), full((D, D)), full((D, D)),
            full((1, D)),
        ],
        out_specs=[pl.BlockSpec((BN, D), lambda i: (i, 0))] * 2,
        out_shape=[jax.ShapeDtypeStruct((N, D), F32)] * 2,
    )(pos2, x2, h, enc_W, enc_b, w1a, w1b, b1)


def _t2_call(z, h, a0, a1, w2a, w2b, b2, dw1a, dw1b):
    def body(z_r, h_r, a0_r, a1_r, w2a_r, w2b_r, b2_r, dw1a_r, dw1b_r,
             hn_r, A_r, B_r):
        acc = _dot(z_r[...], w2a_r[...]) + _dot(h_r[...], w2b_r[...])
        hn = jnp.maximum(acc + b2_r[...] + a0_r[...] + a1_r[...], 0.0)
        hn_r[...] = hn
        A_r[...] = _dot(hn, dw1a_r[...])
        B_r[...] = _dot(hn, dw1b_r[...])

    full = lambda shape: pl.BlockSpec(shape, lambda i: (0, 0))
    blk = pl.BlockSpec((BN, D), lambda i: (i, 0))
    return pl.pallas_call(
        body,
        grid=(N // BN,),
        in_specs=[blk, blk, blk, blk,
                  full((D, D)), full((D, D)), full((1, D)),
                  full((D, D)), full((D, D))],
        out_specs=[blk] * 3,
        out_shape=[jax.ShapeDtypeStruct((N, D), F32)] * 3,
    )(z, h, a0, a1, w2a, w2b, b2, dw1a, dw1b)


def _t3_call(ga, gb, db1, dw2, db2):
    def body(ga_r, gb_r, db1_r, dw2_r, db2_r, out_r):
        x = jnp.maximum(ga_r[...] + gb_r[...] + db1_r[...], 0.0)
        u = _dot(x, dw2_r[...]) + db2_r[...]
        out_r[...] = jax.nn.sigmoid(u)

    full = lambda shape: pl.BlockSpec(shape, lambda i: (0, 0))
    blk = pl.BlockSpec((BE, D), lambda i: (i, 0))
    return pl.pallas_call(
        body,
        grid=(E // BE,),
        in_specs=[blk, blk, full((1, D)), full((D, 1)), full((1, 1))],
        out_specs=pl.BlockSpec((BE, 1), lambda i: (i, 0)),
        out_shape=jax.ShapeDtypeStruct((E, 1), F32),
    )(ga, gb, db1, dw2, db2)


def _t4_call(cnt):
    def body(cnt_r, out_r):
        ssum = jnp.sum(cnt_r[...], axis=0, keepdims=True)
        out_r[...] = jnp.where(ssum > 0.0, 1.0, 0.0).astype(F32)

    return pl.pallas_call(
        body,
        out_shape=jax.ShapeDtypeStruct((1, N), F32),
    )(cnt)


def _t5e_call(a1, a2, a3, edges_r, h1, h2, h3):
    eps = 1e-7

    def body(a1_r, a2_r, a3_r, e_r, h1_r, h2_r, h3_r, acc_r):
        e = e_r[...]

        def bce_sum(a_blk, t_blk):
            p = jnp.clip(a_blk, eps, 1.0 - eps)
            return jnp.sum(-(t_blk * jnp.log(p)
                             + (1.0 - t_blk) * jnp.log(1.0 - p)))

        lx = bce_sum(a3_r[...], e)
        lh = (bce_sum(a1_r[...], h1_r[...])
              + bce_sum(a2_r[...], h2_r[...])
              + bce_sum(a3_r[...], h3_r[...]))

        def cnt_match(a_blk):
            pred = jnp.where(a_blk > 0.4, 1.0, 0.0).astype(F32)
            return jnp.sum(jnp.where(pred == e, 1.0, 0.0))

        ce = cnt_match(a1_r[...]) + cnt_match(a2_r[...]) + cnt_match(a3_r[...])
        lanei = lax.broadcasted_iota(I32, (1, 128), 1)
        vec = (jnp.where(lanei == 0, lx, 0.0)
               + jnp.where(lanei == 1, lh, 0.0)
               + jnp.where(lanei == 2, ce, 0.0))
        acc_r[...] = vec

    return pl.pallas_call(
        body,
        out_shape=jax.ShapeDtypeStruct((1, 128), F32),
    )(a1, a2, a3, edges_r, h1, h2, h3)


def _t5n_call(bp, cp, pi_r, reach_r, rh_r):
    def body(bp_r, cp_r, pi_rr, re_r, rh_rr, acc_r):
        bpv = bp_r[...]
        best = jnp.max(bpv, axis=0, keepdims=True)
        cand = jnp.max(jnp.where(bpv == best, cp_r[...], -1),
                       axis=0, keepdims=True)
        col = lax.broadcasted_iota(I32, (1, N), 1)
        parents = jnp.where(cand >= 0, cand, col)
        cpar = jnp.sum(jnp.where(parents == pi_rr[...], 1.0, 0.0))
        crch = jnp.sum(jnp.where(jnp.round(re_r[...]) == rh_rr[...], 1.0, 0.0))
        lanei = lax.broadcasted_iota(I32, (1, 128), 1)
        vec = (jnp.where(lanei == 0, cpar, 0.0)
               + jnp.where(lanei == 1, crch, 0.0))
        acc_r[...] = vec

    return pl.pallas_call(
        body,
        out_shape=jax.ShapeDtypeStruct((1, 128), F32),
    )(bp, cp, pi_r, reach_r, rh_r)


def kernel(pos, s, edges, edges_h, reach_h, edge_index, pi,
           enc_W, enc_b, mp_W1, mp_b1, mp_W2, mp_b2,
           dec_W1, dec_b1, dec_W2, dec_b2):
    src = edge_index[0]
    dst = edge_index[1]
    hints = edges_h[1:]
    T = hints.shape[0]

    w1a, w1b = mp_W1[:D], mp_W1[D:]
    w2a, w2b = mp_W2[:D], mp_W2[D:]
    dw1a, dw1b = dec_W1[:D], dec_W1[D:]
    enc_b_r = enc_b.reshape(1, D)
    b1_r = mp_b1.reshape(1, D)
    b2_r = mp_b2.reshape(1, D)
    db1_r = dec_b1.reshape(1, D)
    db2_r = dec_b2.reshape(1, 1)

    pos2 = pos.reshape(N, 1).astype(F32)
    x2 = s.reshape(N, 1).astype(F32)
    zeros_nd = jnp.zeros((N, D), F32)
    zeros_n = jnp.zeros((N,), F32)
    neg1f = jnp.full((N,), -1.0, F32)
    neg1i = jnp.full((N,), -1, I32)

    h = zeros_nd
    alphas = []
    reach_row = None
    for _ in range(T):
        z, m = _t1_call(pos2, x2, h, enc_W, enc_b_r, w1a, w1b, b1_r)
        aggp = _k1_call(m, src, dst, zeros_nd)
        h, a_mat, b_mat = _t2_call(z, h, aggp[0], aggp[1],
                                   w2a, w2b, b2_r, dw1a, dw1b)
        ga, gb = _k2_call(a_mat, b_mat, src, dst)
        alpha = _t3_call(ga, gb, db1_r, dec_W2, db2_r).reshape(E)
        alphas.append(alpha)
        cntp = _k3_call(alpha, src, dst, zeros_n)
        reach_row = _t4_call(cntp.reshape(NW, N))
        x2 = reach_row.reshape(N, 1)

    bp, cp = _k4_call(alphas[-1], src, dst, neg1f, neg1i)
    bp = bp.reshape(NW, N)
    cp = cp.reshape(NW, N)
    acc_e = _t5e_call(alphas[0].reshape(1, E), alphas[1].reshape(1, E),
                      alphas[2].reshape(1, E), edges.reshape(1, E),
                      hints[0].reshape(1, E), hints[1].reshape(1, E),
                      hints[2].reshape(1, E))
    acc_n = _t5n_call(bp, cp, pi.reshape(1, N), reach_row,
                      reach_h[-1].reshape(1, N))

    loss_x = acc_e[0, 0] / E
    loss_h = acc_e[0, 1] / E
    edges_err = acc_e[0, 2] / (E * T) * 100.0
    reach_err = acc_n[0, 1] / N * 100.0
    parents_err = acc_n[0, 0] / N
    return jnp.stack([loss_x, loss_h, edges_err, reach_err, parents_err])
